# baseline (speedup 1.0000x reference)
import functools

import jax
import jax.numpy as jnp
from jax import lax
from jax.experimental import pallas as pl
from jax.experimental.pallas import tpu as pltpu
from jax.experimental.pallas import tpu_sc as plsc

EMBED = 64
LANES = 16
CHUNK = 128
SEQ = 200
BATCH = 1024
TOTAL = SEQ * BATCH
NCHUNKS = TOTAL // CHUNK
VOCAB = 1000000
PAIRS = VOCAB // 2

SLAB = 256
SLABP = SLAB + 1
NSLABS_FULL = VOCAB // SLAB
TAIL = VOCAB - NSLABS_FULL * SLAB
NSLABS = NSLABS_FULL + (1 if TAIL else 0)


def _build_kernels():
    info = plsc.get_sparse_core_info()
    nc, ns = info.num_cores, info.num_subcores
    nw = nc * ns
    mesh = plsc.VectorSubcoreMesh(core_axis_name="c", subcore_axis_name="s")
    params = pltpu.CompilerParams(needs_layout_passes=False)

    k1_iters = (NSLABS_FULL + nw - 1) // nw

    @functools.partial(
        pl.kernel,
        mesh=mesh,
        compiler_params=params,
        out_type=jax.ShapeDtypeStruct((PAIRS, 2 * EMBED), jnp.float32),
        scratch_types=[
            pltpu.VMEM((EMBED, SLABP), jnp.float32),
            pltpu.VMEM((SLAB // 2, 2 * EMBED), jnp.float32),
            pltpu.VMEM((TAIL // 2, 2 * EMBED), jnp.float32),
            pltpu.SemaphoreType.DMA,
        ],
    )
    def transpose_kernel(tabt_hbm, tail2_hbm, out_hbm, slab_v, out_v,
                         tail_v, sem):
        wid = lax.axis_index("s") * nc + lax.axis_index("c")
        rr = [lax.iota(jnp.int32, LANES) + (j * LANES)
              for j in range(EMBED // LANES)]

        @pl.when(wid == 0)
        def _copy_tail():
            pltpu.sync_copy(tail2_hbm, tail_v)
            pltpu.sync_copy(tail_v, out_hbm.at[pl.ds(PAIRS - TAIL // 2,
                                                     TAIL // 2)])

        def do_slab(sidx, cols):
            c0 = pl.multiple_of(sidx * SLAB, SLAB)
            pltpu.sync_copy(tabt_hbm.at[:, pl.ds(c0, cols)],
                            slab_v.at[:, pl.ds(0, cols)])

            def kk_body(kk, carry):
                for e in range(2):
                    cc = jnp.full((LANES,), 2 * kk + e, jnp.int32)
                    for j in range(EMBED // LANES):
                        vals = plsc.load_gather(slab_v, [rr[j], cc])
                        out_v[kk, pl.ds(e * EMBED + j * LANES, LANES)] = vals
                return carry

            lax.fori_loop(0, cols // 2, kk_body, 0)
            k0 = pl.multiple_of(c0 // 2, SLAB // 2)
            pltpu.sync_copy(out_v.at[pl.ds(0, cols // 2)],
                            out_hbm.at[pl.ds(k0, cols // 2)])

        def slab_body(i, carry):
            sidx = wid + i * nw

            @pl.when(sidx < NSLABS_FULL)
            def _full():
                do_slab(sidx, SLAB)

            return carry

        lax.fori_loop(0, k1_iters, slab_body, 0)

    per_w = NCHUNKS // nw

    @functools.partial(
        pl.kernel,
        mesh=mesh,
        compiler_params=params,
        out_type=jax.ShapeDtypeStruct((TOTAL, EMBED), jnp.float32),
        scratch_types=[
            pltpu.VMEM((CHUNK,), jnp.int32),
            pltpu.VMEM((CHUNK,), jnp.int32),
            pltpu.VMEM((CHUNK, 2 * EMBED), jnp.float32),
            pltpu.VMEM((CHUNK, EMBED), jnp.float32),
            pltpu.VMEM((2 * EMBED,), jnp.float32),
            pltpu.VMEM((CHUNK,), jnp.float32),
            pltpu.SemaphoreType.DMA,
        ],
    )
    def gather_kernel(ids_hbm, tab2_hbm, pos_hbm, out_hbm,
                      idx_v, idx2_v, rows_v, out_v, pos_v, parf_v, sem):
        wid = lax.axis_index("s") * nc + lax.axis_index("c")

        def chunk_body(i, carry):
            c = wid * per_w + i
            s_pos = c >> 3
            cb = pl.multiple_of(c * CHUNK, CHUNK)
            pb = pl.multiple_of(s_pos * (2 * EMBED), 2 * EMBED)
            pltpu.sync_copy(ids_hbm.at[pl.ds(cb, CHUNK)], idx_v)
            pltpu.sync_copy(pos_hbm.at[pl.ds(pb, 2 * EMBED)], pos_v)
            for k in range(CHUNK // LANES):
                v = idx_v[pl.ds(k * LANES, LANES)]
                idx2_v[pl.ds(k * LANES, LANES)] = v >> 1
                parf_v[pl.ds(k * LANES, LANES)] = (v & 1).astype(
                    jnp.float32) * 0.8
            pltpu.async_copy(tab2_hbm.at[idx2_v], rows_v, sem).wait()
            pk = [pos_v[pl.ds(j * LANES, LANES)] * 0.2
                  for j in range(EMBED // LANES)]

            def row_body(r, rcarry):
                q = plsc.load_gather(parf_v, [jnp.full((LANES,), r,
                                                       jnp.int32)])
                for j in range(EMBED // LANES):
                    e = rows_v[r, pl.ds(j * LANES, LANES)]
                    o = rows_v[r, pl.ds(EMBED + j * LANES, LANES)]
                    out_v[r, pl.ds(j * LANES, LANES)] = (
                        e * 0.8 + (o - e) * q + pk[j])
                return rcarry

            lax.fori_loop(0, CHUNK, row_body, 0)
            pltpu.sync_copy(out_v, out_hbm.at[pl.ds(cb, CHUNK)])
            return carry

        lax.fori_loop(0, per_w, chunk_body, 0)

    return transpose_kernel, gather_kernel


def kernel(input_ids, table, pos_embedding):
    k1, k2 = _build_kernels()
    ids1d = input_ids.reshape(TOTAL).astype(jnp.int32)
    posp = jnp.pad(pos_embedding[:SEQ], ((0, 0), (0, EMBED))).reshape(
        SEQ * 2 * EMBED)
    tail2 = table[VOCAB - TAIL:].reshape(TAIL // 2, 2 * EMBED)
    tab2 = k1(table.T, tail2)
    out = k2(ids1d, tab2, posp)
    return out.reshape(SEQ, BATCH, EMBED)


# speedup vs baseline: 2.7109x; 2.7109x over previous
import functools

import jax
import jax.numpy as jnp
from jax import lax
from jax.experimental import pallas as pl
from jax.experimental.pallas import tpu as pltpu
from jax.experimental.pallas import tpu_sc as plsc

EMBED = 64
LANES = 16
CHUNK = 128
SEQ = 200
BATCH = 1024
TOTAL = SEQ * BATCH
NCHUNKS = TOTAL // CHUNK


def _build_sc_kernel():
    info = plsc.get_sparse_core_info()
    nc, ns = info.num_cores, info.num_subcores
    nw = nc * ns
    per_w = NCHUNKS // nw

    mesh = plsc.VectorSubcoreMesh(core_axis_name="c", subcore_axis_name="s")

    @functools.partial(
        pl.kernel,
        mesh=mesh,
        compiler_params=pltpu.CompilerParams(use_tc_tiling_on_sc=False),
        out_type=jax.ShapeDtypeStruct((TOTAL, EMBED), jnp.float32),
        scratch_types=[
            pltpu.VMEM((2, CHUNK), jnp.int32),
            pltpu.VMEM((2, CHUNK, EMBED), jnp.float32),
            pltpu.VMEM((2, CHUNK, EMBED), jnp.float32),
            pltpu.VMEM((2, EMBED), jnp.float32),
            pltpu.SemaphoreType.DMA,
            pltpu.SemaphoreType.DMA,
            pltpu.SemaphoreType.DMA,
            pltpu.SemaphoreType.DMA,
        ],
    )
    def sc_kernel(ids_hbm, table_hbm, pos_hbm, out_hbm,
                  idx_v, rows_v, out_v, pos_v, sg0, sg1, so0, so1):
        wid = lax.axis_index("s") * nc + lax.axis_index("c")
        base = wid * per_w
        sg = (sg0, sg1)
        so = (so0, so1)

        def fetch(i, buf):
            c = base + i
            pltpu.sync_copy(ids_hbm.at[c], idx_v.at[buf])
            pltpu.sync_copy(pos_hbm.at[c >> 3], pos_v.at[buf])
            pltpu.async_copy(table_hbm.at[idx_v.at[buf]], rows_v.at[buf],
                             sg[buf])

        def blend(buf):
            pk = [pos_v[buf, pl.ds(j * LANES, LANES)] * 0.2
                  for j in range(EMBED // LANES)]

            def row_body(r, carry):
                for j in range(EMBED // LANES):
                    v = rows_v[buf, r, pl.ds(j * LANES, LANES)]
                    out_v[buf, r, pl.ds(j * LANES, LANES)] = v * 0.8 + pk[j]
                return carry

            lax.fori_loop(0, CHUNK, row_body, 0)

        def wait_gather(buf):
            pltpu.make_async_copy(table_hbm.at[idx_v.at[buf]],
                                  rows_v.at[buf], sg[buf]).wait()

        def wait_out(buf):
            pltpu.make_async_copy(out_v.at[buf],
                                  out_hbm.at[pl.ds(0, CHUNK)],
                                  so[buf]).wait()

        def store(i, buf):
            c = base + i
            pltpu.async_copy(out_v.at[buf],
                             out_hbm.at[pl.ds(c * CHUNK, CHUNK)], so[buf])

        fetch(0, 0)
        fetch(1, 1)

        def step(t, carry):
            for b in range(2):
                i = 2 * t + b
                wait_gather(b)

                @pl.when(t > 0)
                def _drain():
                    wait_out(b)

                blend(b)
                store(i, b)

                @pl.when(i + 2 < per_w)
                def _prefetch():
                    fetch(i + 2, b)

            return carry

        lax.fori_loop(0, per_w // 2, step, 0)
        wait_out(0)
        wait_out(1)

    return sc_kernel


def kernel(input_ids, table, pos_embedding):
    ids2d = input_ids.reshape(NCHUNKS, CHUNK).astype(jnp.int32)
    out = _build_sc_kernel()(ids2d, table, pos_embedding)
    return out.reshape(SEQ, BATCH, EMBED)


# speedup vs baseline: 2.7156x; 1.0017x over previous
import functools

import jax
import jax.numpy as jnp
from jax import lax
from jax.experimental import pallas as pl
from jax.experimental.pallas import tpu as pltpu
from jax.experimental.pallas import tpu_sc as plsc

EMBED = 64
LANES = 16
CHUNK = 128
SEQ = 200
BATCH = 1024
TOTAL = SEQ * BATCH
NCHUNKS = TOTAL // CHUNK


def _build_sc_kernel():
    info = plsc.get_sparse_core_info()
    nc, ns = info.num_cores, info.num_subcores
    nw = nc * ns
    per_w = NCHUNKS // nw

    mesh = plsc.VectorSubcoreMesh(core_axis_name="c", subcore_axis_name="s")

    @functools.partial(
        pl.kernel,
        mesh=mesh,
        compiler_params=pltpu.CompilerParams(use_tc_tiling_on_sc=False),
        out_type=jax.ShapeDtypeStruct((SEQ, BATCH, EMBED), jnp.float32),
        scratch_types=[
            pltpu.VMEM((2, CHUNK), jnp.int32),
            pltpu.VMEM((2, CHUNK, EMBED), jnp.float32),
            pltpu.VMEM((2, CHUNK, EMBED), jnp.float32),
            pltpu.VMEM((2, EMBED), jnp.float32),
            pltpu.SemaphoreType.DMA,
            pltpu.SemaphoreType.DMA,
            pltpu.SemaphoreType.DMA,
            pltpu.SemaphoreType.DMA,
        ],
    )
    def sc_kernel(ids_hbm, table_hbm, pos_hbm, out_hbm,
                  idx_v, rows_v, out_v, pos_v, sg0, sg1, so0, so1):
        wid = lax.axis_index("s") * nc + lax.axis_index("c")
        base = wid * per_w
        sg = (sg0, sg1)
        so = (so0, so1)

        def fetch(i, buf):
            c = base + i
            pltpu.sync_copy(ids_hbm.at[c], idx_v.at[buf])
            pltpu.sync_copy(pos_hbm.at[c >> 3], pos_v.at[buf])
            pltpu.async_copy(table_hbm.at[idx_v.at[buf]], rows_v.at[buf],
                             sg[buf])

        def blend(buf):
            pk = [pos_v[buf, pl.ds(j * LANES, LANES)] * 0.2
                  for j in range(EMBED // LANES)]

            def row_body(r, carry):
                for j in range(EMBED // LANES):
                    v = rows_v[buf, r, pl.ds(j * LANES, LANES)]
                    out_v[buf, r, pl.ds(j * LANES, LANES)] = v * 0.8 + pk[j]
                return carry

            lax.fori_loop(0, CHUNK, row_body, 0)

        def wait_gather(buf):
            pltpu.make_async_copy(table_hbm.at[idx_v.at[buf]],
                                  rows_v.at[buf], sg[buf]).wait()

        def wait_out(buf):
            pltpu.make_async_copy(out_v.at[buf],
                                  out_hbm.at[0, pl.ds(0, CHUNK)],
                                  so[buf]).wait()

        def store(i, buf):
            c = base + i
            pltpu.async_copy(out_v.at[buf],
                             out_hbm.at[c >> 3, pl.ds((c & 7) * CHUNK, CHUNK)],
                             so[buf])

        fetch(0, 0)
        fetch(1, 1)

        def step(t, carry):
            for b in range(2):
                i = 2 * t + b
                wait_gather(b)

                @pl.when(t > 0)
                def _drain():
                    wait_out(b)

                blend(b)
                store(i, b)

                @pl.when(i + 2 < per_w)
                def _prefetch():
                    fetch(i + 2, b)

            return carry

        lax.fori_loop(0, per_w // 2, step, 0)
        wait_out(0)
        wait_out(1)

    return sc_kernel


def kernel(input_ids, table, pos_embedding):
    ids2d = input_ids.reshape(NCHUNKS, CHUNK).astype(jnp.int32)
    return _build_sc_kernel()(ids2d, table, pos_embedding)
